# baseline (speedup 1.0000x reference)
import functools

import jax
import jax.numpy as jnp
from jax import lax
from jax.experimental import pallas as pl
from jax.experimental.pallas import tpu as pltpu
from jax.experimental.pallas import tpu_sc as plsc

N = 10000
E = 320000
D_IN = 128
D_H = 64

NC = 2
NS = 16
NW = NC * NS

EPW = E // NW
CHUNK = 80
NCH = 128
EPW_PAD = NCH * CHUNK
NBUF = 4
RPT = 632
N_PAD = NS * RPT

BLK = 2000


def _agg_body(with_counts, *refs):
    if with_counts:
        (z_hbm, src_hbm, dst_hbm, zrow_hbm, zcnt_hbm,
         out_hbm, cnt_hbm,
         src_v, dst_v, rows_v, ones_v,
         acc_sh, cnt_sh, sem_g, sem_s, sem_c) = refs
    else:
        (z_hbm, src_hbm, dst_hbm, zrow_hbm,
         out_hbm,
         src_v, dst_v, rows_v,
         acc_sh, sem_g, sem_s) = refs

    c = lax.axis_index("c")
    s = lax.axis_index("s")
    wid = c * NS + s

    pltpu.sync_copy(src_hbm.at[wid], src_v)
    pltpu.sync_copy(dst_hbm.at[wid], dst_v)

    pltpu.sync_copy(zrow_hbm, acc_sh.at[pl.ds(s * RPT, RPT)])
    if with_counts:
        pltpu.sync_copy(zcnt_hbm, cnt_sh.at[pl.ds(s * RPT, RPT)])

        def _fill_ones(i, carry):
            ones_v[i, :] = jnp.ones((16,), jnp.float32)
            return carry
        lax.fori_loop(0, CHUNK, _fill_ones, 0)

    plsc.subcore_barrier()

    def g_start(cidx, b):
        pltpu.async_copy(z_hbm.at[src_v.at[cidx]], rows_v.at[b], sem_g.at[b])

    def g_wait(b):
        pltpu.make_async_copy(z_hbm.at[src_v.at[0]], rows_v.at[b],
                              sem_g.at[b]).wait()

    def s_start(cidx, b):
        pltpu.async_copy(rows_v.at[b], acc_sh.at[dst_v.at[cidx]],
                         sem_s.at[b], add=True)
        if with_counts:
            pltpu.async_copy(ones_v, cnt_sh.at[dst_v.at[cidx]],
                             sem_c.at[b], add=True)

    def s_wait(b):
        pltpu.make_async_copy(rows_v.at[b], acc_sh.at[dst_v.at[0]],
                              sem_s.at[b]).wait()
        if with_counts:
            pltpu.make_async_copy(ones_v, cnt_sh.at[dst_v.at[0]],
                                  sem_c.at[b]).wait()

    for b in range(NBUF):
        g_start(b, b)
    for b in range(NBUF):
        g_wait(b)
        s_start(b, b)
        if b >= 2:
            s_wait(b - 2)
            g_start(b + 2, b - 2)

    def _loop(g, carry):
        for b in range(NBUF):
            cidx = NBUF * g + b
            g_wait(b)
            s_start(cidx, b)
            b2 = (b + 2) % NBUF
            s_wait(b2)

            @pl.when(cidx + 2 < NCH)
            def _():
                g_start(cidx + 2, b2)
        return carry
    lax.fori_loop(1, NCH // NBUF, _loop, 0)
    s_wait(2)
    s_wait(3)

    plsc.subcore_barrier()

    pltpu.sync_copy(acc_sh.at[pl.ds(s * RPT, RPT)],
                    out_hbm.at[c, pl.ds(s * RPT, RPT)])
    if with_counts:
        pltpu.sync_copy(cnt_sh.at[pl.ds(s * RPT, RPT)],
                        cnt_hbm.at[c, pl.ds(s * RPT, RPT)])


def _make_agg(with_counts):
    mesh = plsc.VectorSubcoreMesh(core_axis_name="c", subcore_axis_name="s")
    out_type = [jax.ShapeDtypeStruct((NC, N_PAD, D_H), jnp.float32)]
    scratch = [
        pltpu.VMEM((NCH, CHUNK), jnp.int32),
        pltpu.VMEM((NCH, CHUNK), jnp.int32),
        pltpu.VMEM((NBUF, CHUNK, D_H), jnp.float32),
    ]
    if with_counts:
        out_type.append(jax.ShapeDtypeStruct((NC, N_PAD, 16), jnp.float32))
        scratch.append(pltpu.VMEM((CHUNK, 16), jnp.float32))
    scratch.append(pltpu.VMEM_SHARED((N_PAD, D_H), jnp.float32))
    if with_counts:
        scratch.append(pltpu.VMEM_SHARED((N_PAD, 16), jnp.float32))
    scratch += [pltpu.SemaphoreType.DMA((NBUF,)), pltpu.SemaphoreType.DMA((NBUF,))]
    if with_counts:
        scratch.append(pltpu.SemaphoreType.DMA((NBUF,)))
    return pl.kernel(
        functools.partial(_agg_body, with_counts),
        out_type=out_type,
        mesh=mesh,
        scratch_types=scratch,
        compiler_params=pltpu.CompilerParams(use_tc_tiling_on_sc=False),
    )


_agg_with_counts = _make_agg(True)
_agg_no_counts = _make_agg(False)


def _mm_body(x_ref, w_ref, o_ref):
    o_ref[...] = jnp.dot(x_ref[...], w_ref[...],
                         preferred_element_type=jnp.float32)


_mm = pl.pallas_call(
    _mm_body,
    grid=(N // BLK,),
    in_specs=[
        pl.BlockSpec((BLK, D_IN), lambda i: (i, 0)),
        pl.BlockSpec((D_IN, D_H), lambda i: (0, 0)),
    ],
    out_specs=pl.BlockSpec((BLK, D_H), lambda i: (i, 0)),
    out_shape=jax.ShapeDtypeStruct((N, D_H), jnp.float32),
)


def _mid_body(p_ref, c_ref, x_ref, w1r_ref, b1l_ref, w2l_ref,
              h1_ref, z2_ref):
    cnt = jnp.maximum(c_ref[0, :, :1] + c_ref[1, :, :1], 1.0)
    agg = (p_ref[0] + p_ref[1]) / cnt
    h1 = jnp.maximum(
        agg + b1l_ref[...]
        + jnp.dot(x_ref[...], w1r_ref[...], preferred_element_type=jnp.float32),
        0.0)
    h1_ref[...] = h1
    z2_ref[...] = jnp.dot(h1, w2l_ref[...], preferred_element_type=jnp.float32)


_mid = pl.pallas_call(
    _mid_body,
    grid=(N // BLK,),
    in_specs=[
        pl.BlockSpec((2, BLK, D_H), lambda i: (0, i, 0)),
        pl.BlockSpec((2, BLK, 16), lambda i: (0, i, 0)),
        pl.BlockSpec((BLK, D_IN), lambda i: (i, 0)),
        pl.BlockSpec((D_IN, D_H), lambda i: (0, 0)),
        pl.BlockSpec((1, D_H), lambda i: (0, 0)),
        pl.BlockSpec((D_H, D_H), lambda i: (0, 0)),
    ],
    out_specs=[
        pl.BlockSpec((BLK, D_H), lambda i: (i, 0)),
        pl.BlockSpec((BLK, D_H), lambda i: (i, 0)),
    ],
    out_shape=[
        jax.ShapeDtypeStruct((N, D_H), jnp.float32),
        jax.ShapeDtypeStruct((N, D_H), jnp.float32),
    ],
)


def _fin_body(p_ref, c_ref, h1_ref, w2r_ref, b2l_ref,
              wp1_ref, bp1_ref, wp2_ref, bp2_ref, o_ref):
    cnt = jnp.maximum(c_ref[0, :, :1] + c_ref[1, :, :1], 1.0)
    agg = (p_ref[0] + p_ref[1]) / cnt
    h2 = jnp.maximum(
        agg + b2l_ref[...]
        + jnp.dot(h1_ref[...], w2r_ref[...],
                  preferred_element_type=jnp.float32),
        0.0)
    p = jnp.maximum(
        jnp.dot(h2, wp1_ref[...], preferred_element_type=jnp.float32)
        + bp1_ref[...],
        0.0)
    o_ref[...] = (jnp.dot(p, wp2_ref[...], preferred_element_type=jnp.float32)
                  + bp2_ref[...])


_fin = pl.pallas_call(
    _fin_body,
    grid=(N // BLK,),
    in_specs=[
        pl.BlockSpec((2, BLK, D_H), lambda i: (0, i, 0)),
        pl.BlockSpec((2, BLK, 16), lambda i: (0, i, 0)),
        pl.BlockSpec((BLK, D_H), lambda i: (i, 0)),
        pl.BlockSpec((D_H, D_H), lambda i: (0, 0)),
        pl.BlockSpec((1, D_H), lambda i: (0, 0)),
        pl.BlockSpec((D_H, D_H // 2), lambda i: (0, 0)),
        pl.BlockSpec((1, D_H // 2), lambda i: (0, 0)),
        pl.BlockSpec((D_H // 2, 1), lambda i: (0, 0)),
        pl.BlockSpec((1, 1), lambda i: (0, 0)),
    ],
    out_specs=pl.BlockSpec((BLK, 1), lambda i: (i, 0)),
    out_shape=jax.ShapeDtypeStruct((N, 1), jnp.float32),
)


def kernel(x, edge_index, W1l, b1l, W1r, W2l, b2l, W2r, Wp1, bp1, Wp2, bp2):
    pad = ((0, 0), (0, EPW_PAD - EPW))
    src = jnp.pad(edge_index[0].reshape(NW, EPW), pad,
                  constant_values=0).reshape(NW, NCH, CHUNK)
    dst = jnp.pad(edge_index[1].reshape(NW, EPW), pad,
                  constant_values=N).reshape(NW, NCH, CHUNK)
    zrow = jnp.zeros((RPT, D_H), jnp.float32)
    zcnt = jnp.zeros((RPT, 16), jnp.float32)

    b1l2 = b1l.reshape(1, D_H)
    b2l2 = b2l.reshape(1, D_H)
    bp12 = bp1.reshape(1, D_H // 2)
    bp22 = bp2.reshape(1, 1)

    z1 = _mm(x, W1l)
    p1, cnt = _agg_with_counts(z1, src, dst, zrow, zcnt)
    h1, z2 = _mid(p1, cnt, x, W1r, b1l2, W2l)
    (p2,) = _agg_no_counts(z2, src, dst, zrow)
    return _fin(p2, cnt, h1, W2r, b2l2, Wp1, bp12, Wp2, bp22)


# speedup vs baseline: 1.9327x; 1.9327x over previous
import functools

import jax
import jax.numpy as jnp
from jax import lax
from jax.experimental import pallas as pl
from jax.experimental.pallas import tpu as pltpu
from jax.experimental.pallas import tpu_sc as plsc

N = 10000
E = 320000
D_IN = 128
D_H = 64

NC = 2
NS = 16
NW = NC * NS

EPW = E // NW
CHUNK = 80
NCH = 125
EPW_PAD = NCH * CHUNK
RPT = 632
N_PAD = NS * RPT

BLK = 2000


def _agg_body(with_counts, *refs):
    if with_counts:
        (z_hbm, src_hbm, dst_hbm, zrow_hbm, zcnt_hbm,
         out_hbm, cnt_hbm,
         src_v, dst_v, rows0, rows1, ones_v,
         z_sh, acc_sh, cnt_sh, sem0, sem1) = refs
    else:
        (z_hbm, src_hbm, dst_hbm, zrow_hbm,
         out_hbm,
         src_v, dst_v, rows0, rows1,
         z_sh, acc_sh, sem0, sem1) = refs

    c = lax.axis_index("c")
    s = lax.axis_index("s")
    wid = c * NS + s

    pltpu.sync_copy(src_hbm.at[wid], src_v)
    pltpu.sync_copy(dst_hbm.at[wid], dst_v)

    pltpu.sync_copy(zrow_hbm, acc_sh.at[pl.ds(s * RPT, RPT)])
    if with_counts:
        pltpu.sync_copy(zcnt_hbm, cnt_sh.at[pl.ds(s * RPT, RPT)])

        def _fill_ones(i, carry):
            ones_v[i, :] = jnp.ones((16,), jnp.float32)
            return carry
        lax.fori_loop(0, CHUNK, _fill_ones, 0)

    @pl.when(s == 0)
    def _():
        pltpu.sync_copy(z_hbm, z_sh)

    plsc.subcore_barrier()

    pltpu.async_copy(z_sh.at[src_v.at[0]], rows0, sem0)
    pltpu.async_copy(z_sh.at[src_v.at[1]], rows1, sem1)

    def _step(cidx, rows, sem, last=False):
        pltpu.make_async_copy(z_sh.at[src_v.at[0]], rows, sem).wait()
        pltpu.sync_copy(rows, acc_sh.at[dst_v.at[cidx]], add=True)
        if with_counts:
            pltpu.sync_copy(ones_v, cnt_sh.at[dst_v.at[cidx]], add=True)
        if last:
            return

        @pl.when(cidx + 2 < NCH)
        def _():
            pltpu.async_copy(z_sh.at[src_v.at[cidx + 2]], rows, sem)

    def _loop(g, carry):
        _step(2 * g, rows0, sem0)
        _step(2 * g + 1, rows1, sem1)
        return carry
    lax.fori_loop(0, (NCH - 1) // 2, _loop, 0)
    _step(NCH - 1, rows0, sem0, last=True)

    plsc.subcore_barrier()

    pltpu.sync_copy(acc_sh.at[pl.ds(s * RPT, RPT)],
                    out_hbm.at[c, pl.ds(s * RPT, RPT)])
    if with_counts:
        pltpu.sync_copy(cnt_sh.at[pl.ds(s * RPT, RPT)],
                        cnt_hbm.at[c, pl.ds(s * RPT, RPT)])


def _make_agg(with_counts):
    mesh = plsc.VectorSubcoreMesh(core_axis_name="c", subcore_axis_name="s")
    out_type = [jax.ShapeDtypeStruct((NC, N_PAD, D_H), jnp.float32)]
    scratch = [
        pltpu.VMEM((NCH, CHUNK), jnp.int32),
        pltpu.VMEM((NCH, CHUNK), jnp.int32),
        pltpu.VMEM((CHUNK, D_H), jnp.float32),
        pltpu.VMEM((CHUNK, D_H), jnp.float32),
    ]
    if with_counts:
        out_type.append(jax.ShapeDtypeStruct((NC, N_PAD, 16), jnp.float32))
        scratch.append(pltpu.VMEM((CHUNK, 16), jnp.float32))
    scratch.append(pltpu.VMEM_SHARED((N, D_H), jnp.float32))
    scratch.append(pltpu.VMEM_SHARED((N_PAD, D_H), jnp.float32))
    if with_counts:
        scratch.append(pltpu.VMEM_SHARED((N_PAD, 16), jnp.float32))
    scratch += [pltpu.SemaphoreType.DMA, pltpu.SemaphoreType.DMA]
    return pl.kernel(
        functools.partial(_agg_body, with_counts),
        out_type=out_type,
        mesh=mesh,
        scratch_types=scratch,
        compiler_params=pltpu.CompilerParams(use_tc_tiling_on_sc=False),
    )


_agg_with_counts = _make_agg(True)
_agg_no_counts = _make_agg(False)


def _mm_body(x_ref, w_ref, o_ref):
    o_ref[...] = jnp.dot(x_ref[...], w_ref[...],
                         preferred_element_type=jnp.float32)


_mm = pl.pallas_call(
    _mm_body,
    grid=(N // BLK,),
    in_specs=[
        pl.BlockSpec((BLK, D_IN), lambda i: (i, 0)),
        pl.BlockSpec((D_IN, D_H), lambda i: (0, 0)),
    ],
    out_specs=pl.BlockSpec((BLK, D_H), lambda i: (i, 0)),
    out_shape=jax.ShapeDtypeStruct((N, D_H), jnp.float32),
)


def _mid_body(p_ref, c_ref, x_ref, w1r_ref, b1l_ref, w2l_ref,
              h1_ref, z2_ref):
    cnt = jnp.maximum(c_ref[0, :, :1] + c_ref[1, :, :1], 1.0)
    agg = (p_ref[0] + p_ref[1]) / cnt
    h1 = jnp.maximum(
        agg + b1l_ref[...]
        + jnp.dot(x_ref[...], w1r_ref[...], preferred_element_type=jnp.float32),
        0.0)
    h1_ref[...] = h1
    z2_ref[...] = jnp.dot(h1, w2l_ref[...], preferred_element_type=jnp.float32)


_mid = pl.pallas_call(
    _mid_body,
    grid=(N // BLK,),
    in_specs=[
        pl.BlockSpec((2, BLK, D_H), lambda i: (0, i, 0)),
        pl.BlockSpec((2, BLK, 16), lambda i: (0, i, 0)),
        pl.BlockSpec((BLK, D_IN), lambda i: (i, 0)),
        pl.BlockSpec((D_IN, D_H), lambda i: (0, 0)),
        pl.BlockSpec((1, D_H), lambda i: (0, 0)),
        pl.BlockSpec((D_H, D_H), lambda i: (0, 0)),
    ],
    out_specs=[
        pl.BlockSpec((BLK, D_H), lambda i: (i, 0)),
        pl.BlockSpec((BLK, D_H), lambda i: (i, 0)),
    ],
    out_shape=[
        jax.ShapeDtypeStruct((N, D_H), jnp.float32),
        jax.ShapeDtypeStruct((N, D_H), jnp.float32),
    ],
)


def _fin_body(p_ref, c_ref, h1_ref, w2r_ref, b2l_ref,
              wp1_ref, bp1_ref, wp2_ref, bp2_ref, o_ref):
    cnt = jnp.maximum(c_ref[0, :, :1] + c_ref[1, :, :1], 1.0)
    agg = (p_ref[0] + p_ref[1]) / cnt
    h2 = jnp.maximum(
        agg + b2l_ref[...]
        + jnp.dot(h1_ref[...], w2r_ref[...],
                  preferred_element_type=jnp.float32),
        0.0)
    p = jnp.maximum(
        jnp.dot(h2, wp1_ref[...], preferred_element_type=jnp.float32)
        + bp1_ref[...],
        0.0)
    o_ref[...] = (jnp.dot(p, wp2_ref[...], preferred_element_type=jnp.float32)
                  + bp2_ref[...])


_fin = pl.pallas_call(
    _fin_body,
    grid=(N // BLK,),
    in_specs=[
        pl.BlockSpec((2, BLK, D_H), lambda i: (0, i, 0)),
        pl.BlockSpec((2, BLK, 16), lambda i: (0, i, 0)),
        pl.BlockSpec((BLK, D_H), lambda i: (i, 0)),
        pl.BlockSpec((D_H, D_H), lambda i: (0, 0)),
        pl.BlockSpec((1, D_H), lambda i: (0, 0)),
        pl.BlockSpec((D_H, D_H // 2), lambda i: (0, 0)),
        pl.BlockSpec((1, D_H // 2), lambda i: (0, 0)),
        pl.BlockSpec((D_H // 2, 1), lambda i: (0, 0)),
        pl.BlockSpec((1, 1), lambda i: (0, 0)),
    ],
    out_specs=pl.BlockSpec((BLK, 1), lambda i: (i, 0)),
    out_shape=jax.ShapeDtypeStruct((N, 1), jnp.float32),
)


def kernel(x, edge_index, W1l, b1l, W1r, W2l, b2l, W2r, Wp1, bp1, Wp2, bp2):
    pad = ((0, 0), (0, EPW_PAD - EPW))
    src = jnp.pad(edge_index[0].reshape(NW, EPW), pad,
                  constant_values=0).reshape(NW, NCH, CHUNK)
    dst = jnp.pad(edge_index[1].reshape(NW, EPW), pad,
                  constant_values=N).reshape(NW, NCH, CHUNK)
    zrow = jnp.zeros((RPT, D_H), jnp.float32)
    zcnt = jnp.zeros((RPT, 16), jnp.float32)

    b1l2 = b1l.reshape(1, D_H)
    b2l2 = b2l.reshape(1, D_H)
    bp12 = bp1.reshape(1, D_H // 2)
    bp22 = bp2.reshape(1, 1)

    z1 = _mm(x, W1l)
    p1, cnt = _agg_with_counts(z1, src, dst, zrow, zcnt)
    h1, z2 = _mid(p1, cnt, x, W1r, b1l2, W2l)
    (p2,) = _agg_no_counts(z2, src, dst, zrow)
    return _fin(p2, cnt, h1, W2r, b2l2, Wp1, bp12, Wp2, bp22)


# speedup vs baseline: 2.0490x; 1.0602x over previous
import functools

import jax
import jax.numpy as jnp
from jax import lax
from jax.experimental import pallas as pl
from jax.experimental.pallas import tpu as pltpu
from jax.experimental.pallas import tpu_sc as plsc

N = 10000
E = 320000
D_IN = 128
D_H = 64

NC = 2
NS = 16
NW = NC * NS

EPW = E // NW
CHUNK = 80
NCH = 125
EPW_PAD = NCH * CHUNK
RPT = 632
N_PAD = NS * RPT

BLK = 2000


def _agg_body(with_counts, *refs):
    if with_counts:
        (z_hbm, edge_hbm, zrow_hbm, zcnt_hbm,
         out_hbm, cnt_hbm,
         src_v, dst_v, rows0, rows1, ones_v,
         acc_sh, cnt_sh, sem0, sem1) = refs
    else:
        (z_hbm, edge_hbm, zrow_hbm,
         out_hbm,
         src_v, dst_v, rows0, rows1,
         acc_sh, sem0, sem1) = refs

    c = lax.axis_index("c")
    s = lax.axis_index("s")
    wid = c * NS + s

    pltpu.sync_copy(edge_hbm.at[0, wid], src_v)
    pltpu.sync_copy(edge_hbm.at[1, wid], dst_v)

    pltpu.sync_copy(zrow_hbm, acc_sh.at[pl.ds(s * RPT, RPT)])
    if with_counts:
        pltpu.sync_copy(zcnt_hbm, cnt_sh.at[pl.ds(s * RPT, RPT)])

        def _fill_ones(i, carry):
            ones_v[i, :] = jnp.ones((16,), jnp.float32)
            return carry
        lax.fori_loop(0, CHUNK, _fill_ones, 0)

    plsc.subcore_barrier()

    pltpu.async_copy(z_hbm.at[src_v.at[0]], rows0, sem0)
    pltpu.async_copy(z_hbm.at[src_v.at[1]], rows1, sem1)

    def _step(cidx, rows, sem, last=False):
        pltpu.make_async_copy(z_hbm.at[src_v.at[0]], rows, sem).wait()
        pltpu.sync_copy(rows, acc_sh.at[dst_v.at[cidx]], add=True)
        if with_counts:
            pltpu.sync_copy(ones_v, cnt_sh.at[dst_v.at[cidx]], add=True)
        if last:
            return

        @pl.when(cidx + 2 < NCH)
        def _():
            pltpu.async_copy(z_hbm.at[src_v.at[cidx + 2]], rows, sem)

    def _loop(g, carry):
        _step(2 * g, rows0, sem0)
        _step(2 * g + 1, rows1, sem1)
        return carry
    lax.fori_loop(0, (NCH - 1) // 2, _loop, 0)
    _step(NCH - 1, rows0, sem0, last=True)

    plsc.subcore_barrier()

    pltpu.sync_copy(acc_sh.at[pl.ds(s * RPT, RPT)],
                    out_hbm.at[c, pl.ds(s * RPT, RPT)])
    if with_counts:
        pltpu.sync_copy(cnt_sh.at[pl.ds(s * RPT, RPT)],
                        cnt_hbm.at[c, pl.ds(s * RPT, RPT)])


def _make_agg(with_counts):
    mesh = plsc.VectorSubcoreMesh(core_axis_name="c", subcore_axis_name="s")
    out_type = [jax.ShapeDtypeStruct((NC, N_PAD, D_H), jnp.float32)]
    scratch = [
        pltpu.VMEM((NCH, CHUNK), jnp.int32),
        pltpu.VMEM((NCH, CHUNK), jnp.int32),
        pltpu.VMEM((CHUNK, D_H), jnp.float32),
        pltpu.VMEM((CHUNK, D_H), jnp.float32),
    ]
    if with_counts:
        out_type.append(jax.ShapeDtypeStruct((NC, N_PAD, 16), jnp.float32))
        scratch.append(pltpu.VMEM((CHUNK, 16), jnp.float32))
    scratch.append(pltpu.VMEM_SHARED((N_PAD, D_H), jnp.float32))
    if with_counts:
        scratch.append(pltpu.VMEM_SHARED((N_PAD, 16), jnp.float32))
    scratch += [pltpu.SemaphoreType.DMA, pltpu.SemaphoreType.DMA]
    return pl.kernel(
        functools.partial(_agg_body, with_counts),
        out_type=out_type,
        mesh=mesh,
        scratch_types=scratch,
        compiler_params=pltpu.CompilerParams(use_tc_tiling_on_sc=False),
    )


_agg_with_counts = _make_agg(True)
_agg_no_counts = _make_agg(False)


def _mm_body(x_ref, w_ref, o_ref):
    o_ref[...] = jnp.dot(x_ref[...], w_ref[...],
                         preferred_element_type=jnp.float32)


_mm = pl.pallas_call(
    _mm_body,
    grid=(N // BLK,),
    in_specs=[
        pl.BlockSpec((BLK, D_IN), lambda i: (i, 0)),
        pl.BlockSpec((D_IN, D_H), lambda i: (0, 0)),
    ],
    out_specs=pl.BlockSpec((BLK, D_H), lambda i: (i, 0)),
    out_shape=jax.ShapeDtypeStruct((N, D_H), jnp.float32),
)


def _mid_body(p_ref, c_ref, x_ref, w1r_ref, b1l_ref, w2l_ref,
              h1_ref, z2_ref):
    cnt = jnp.maximum(c_ref[0, :, :1] + c_ref[1, :, :1], 1.0)
    agg = (p_ref[0] + p_ref[1]) / cnt
    h1 = jnp.maximum(
        agg + b1l_ref[...]
        + jnp.dot(x_ref[...], w1r_ref[...], preferred_element_type=jnp.float32),
        0.0)
    h1_ref[...] = h1
    z2_ref[...] = jnp.dot(h1, w2l_ref[...], preferred_element_type=jnp.float32)


_mid = pl.pallas_call(
    _mid_body,
    grid=(N // BLK,),
    in_specs=[
        pl.BlockSpec((2, BLK, D_H), lambda i: (0, i, 0)),
        pl.BlockSpec((2, BLK, 16), lambda i: (0, i, 0)),
        pl.BlockSpec((BLK, D_IN), lambda i: (i, 0)),
        pl.BlockSpec((D_IN, D_H), lambda i: (0, 0)),
        pl.BlockSpec((1, D_H), lambda i: (0, 0)),
        pl.BlockSpec((D_H, D_H), lambda i: (0, 0)),
    ],
    out_specs=[
        pl.BlockSpec((BLK, D_H), lambda i: (i, 0)),
        pl.BlockSpec((BLK, D_H), lambda i: (i, 0)),
    ],
    out_shape=[
        jax.ShapeDtypeStruct((N, D_H), jnp.float32),
        jax.ShapeDtypeStruct((N, D_H), jnp.float32),
    ],
)


def _fin_body(p_ref, c_ref, h1_ref, w2r_ref, b2l_ref,
              wp1_ref, bp1_ref, wp2_ref, bp2_ref, o_ref):
    cnt = jnp.maximum(c_ref[0, :, :1] + c_ref[1, :, :1], 1.0)
    agg = (p_ref[0] + p_ref[1]) / cnt
    h2 = jnp.maximum(
        agg + b2l_ref[...]
        + jnp.dot(h1_ref[...], w2r_ref[...],
                  preferred_element_type=jnp.float32),
        0.0)
    p = jnp.maximum(
        jnp.dot(h2, wp1_ref[...], preferred_element_type=jnp.float32)
        + bp1_ref[...],
        0.0)
    o_ref[...] = (jnp.dot(p, wp2_ref[...], preferred_element_type=jnp.float32)
                  + bp2_ref[...])


_fin = pl.pallas_call(
    _fin_body,
    grid=(N // BLK,),
    in_specs=[
        pl.BlockSpec((2, BLK, D_H), lambda i: (0, i, 0)),
        pl.BlockSpec((2, BLK, 16), lambda i: (0, i, 0)),
        pl.BlockSpec((BLK, D_H), lambda i: (i, 0)),
        pl.BlockSpec((D_H, D_H), lambda i: (0, 0)),
        pl.BlockSpec((1, D_H), lambda i: (0, 0)),
        pl.BlockSpec((D_H, D_H // 2), lambda i: (0, 0)),
        pl.BlockSpec((1, D_H // 2), lambda i: (0, 0)),
        pl.BlockSpec((D_H // 2, 1), lambda i: (0, 0)),
        pl.BlockSpec((1, 1), lambda i: (0, 0)),
    ],
    out_specs=pl.BlockSpec((BLK, 1), lambda i: (i, 0)),
    out_shape=jax.ShapeDtypeStruct((N, 1), jnp.float32),
)


def kernel(x, edge_index, W1l, b1l, W1r, W2l, b2l, W2r, Wp1, bp1, Wp2, bp2):
    edges = edge_index.reshape(2, NW, NCH, CHUNK)
    zrow = jnp.zeros((RPT, D_H), jnp.float32)
    zcnt = jnp.zeros((RPT, 16), jnp.float32)

    b1l2 = b1l.reshape(1, D_H)
    b2l2 = b2l.reshape(1, D_H)
    bp12 = bp1.reshape(1, D_H // 2)
    bp22 = bp2.reshape(1, 1)

    z1 = _mm(x, W1l)
    p1, cnt = _agg_with_counts(z1, edges, zrow, zcnt)
    h1, z2 = _mid(p1, cnt, x, W1r, b1l2, W2l)
    (p2,) = _agg_no_counts(z2, edges, zrow)
    return _fin(p2, cnt, h1, W2r, b2l2, Wp1, bp12, Wp2, bp22)


# speedup vs baseline: 2.1003x; 1.0250x over previous
import functools

import jax
import jax.numpy as jnp
from jax import lax
from jax.experimental import pallas as pl
from jax.experimental.pallas import tpu as pltpu
from jax.experimental.pallas import tpu_sc as plsc

N = 10000
E = 320000
D_IN = 128
D_H = 64

NC = 2
NS = 16
NW = NC * NS

EPW = E // NW
CHUNK = 80
NCH = 125
EPW_PAD = NCH * CHUNK
RPT = 632
N_PAD = NS * RPT

BLK = 2000


def _agg_body(with_counts, *refs):
    if with_counts:
        (z_hbm, edge_hbm, zrow_hbm, zcnt_hbm,
         out_hbm, cnt_hbm,
         src_v, dst_v, rows0, rows1, ones_v,
         acc_sh, cnt_sh, sem0, sem1, semc0, semc1) = refs
    else:
        (z_hbm, edge_hbm, zrow_hbm,
         out_hbm,
         src_v, dst_v, rows0, rows1,
         acc_sh, sem0, sem1) = refs
        semc0 = semc1 = None

    c = lax.axis_index("c")
    s = lax.axis_index("s")
    wid = c * NS + s

    pltpu.sync_copy(edge_hbm.at[0, wid], src_v)
    pltpu.sync_copy(edge_hbm.at[1, wid], dst_v)

    pltpu.sync_copy(zrow_hbm, acc_sh.at[pl.ds(s * RPT, RPT)])
    if with_counts:
        pltpu.sync_copy(zcnt_hbm, cnt_sh.at[pl.ds(s * RPT, RPT)])

        def _fill_ones(i, carry):
            ones_v[i, :] = jnp.ones((16,), jnp.float32)
            return carry
        lax.fori_loop(0, CHUNK, _fill_ones, 0)

    plsc.subcore_barrier()

    pltpu.async_copy(z_hbm.at[src_v.at[0]], rows0, sem0)
    pltpu.async_copy(z_hbm.at[src_v.at[1]], rows1, sem1)

    def _step(cidx, rows, sem, semc, last=False):
        pltpu.make_async_copy(z_hbm.at[src_v.at[0]], rows, sem).wait()
        pltpu.sync_copy(rows, acc_sh.at[dst_v.at[cidx]], add=True)
        if with_counts:
            @pl.when(cidx >= 2)
            def _():
                pltpu.make_async_copy(ones_v, cnt_sh.at[dst_v.at[0]],
                                      semc).wait()
            pltpu.async_copy(ones_v, cnt_sh.at[dst_v.at[cidx]], semc,
                             add=True)
        if last:
            return

        @pl.when(cidx + 2 < NCH)
        def _():
            pltpu.async_copy(z_hbm.at[src_v.at[cidx + 2]], rows, sem)

    def _loop(g, carry):
        _step(2 * g, rows0, sem0, semc0)
        _step(2 * g + 1, rows1, sem1, semc1)
        return carry
    lax.fori_loop(0, (NCH - 1) // 2, _loop, 0)
    _step(NCH - 1, rows0, sem0, semc0, last=True)
    if with_counts:
        pltpu.make_async_copy(ones_v, cnt_sh.at[dst_v.at[0]], semc0).wait()
        pltpu.make_async_copy(ones_v, cnt_sh.at[dst_v.at[0]], semc1).wait()

    plsc.subcore_barrier()

    pltpu.sync_copy(acc_sh.at[pl.ds(s * RPT, RPT)],
                    out_hbm.at[c, pl.ds(s * RPT, RPT)])
    if with_counts:
        pltpu.sync_copy(cnt_sh.at[pl.ds(s * RPT, RPT)],
                        cnt_hbm.at[c, pl.ds(s * RPT, RPT)])


def _make_agg(with_counts):
    mesh = plsc.VectorSubcoreMesh(core_axis_name="c", subcore_axis_name="s")
    out_type = [jax.ShapeDtypeStruct((NC, N_PAD, D_H), jnp.float32)]
    scratch = [
        pltpu.VMEM((NCH, CHUNK), jnp.int32),
        pltpu.VMEM((NCH, CHUNK), jnp.int32),
        pltpu.VMEM((CHUNK, D_H), jnp.float32),
        pltpu.VMEM((CHUNK, D_H), jnp.float32),
    ]
    if with_counts:
        out_type.append(jax.ShapeDtypeStruct((NC, N_PAD, 16), jnp.float32))
        scratch.append(pltpu.VMEM((CHUNK, 16), jnp.float32))
    scratch.append(pltpu.VMEM_SHARED((N_PAD, D_H), jnp.float32))
    if with_counts:
        scratch.append(pltpu.VMEM_SHARED((N_PAD, 16), jnp.float32))
    scratch += [pltpu.SemaphoreType.DMA, pltpu.SemaphoreType.DMA]
    if with_counts:
        scratch += [pltpu.SemaphoreType.DMA, pltpu.SemaphoreType.DMA]
    return pl.kernel(
        functools.partial(_agg_body, with_counts),
        out_type=out_type,
        mesh=mesh,
        scratch_types=scratch,
        compiler_params=pltpu.CompilerParams(use_tc_tiling_on_sc=False),
    )


_agg_with_counts = _make_agg(True)
_agg_no_counts = _make_agg(False)


def _mm_body(x_ref, w_ref, o_ref):
    o_ref[...] = jnp.dot(x_ref[...], w_ref[...],
                         preferred_element_type=jnp.float32)


_mm = pl.pallas_call(
    _mm_body,
    grid=(N // BLK,),
    in_specs=[
        pl.BlockSpec((BLK, D_IN), lambda i: (i, 0)),
        pl.BlockSpec((D_IN, D_H), lambda i: (0, 0)),
    ],
    out_specs=pl.BlockSpec((BLK, D_H), lambda i: (i, 0)),
    out_shape=jax.ShapeDtypeStruct((N, D_H), jnp.float32),
)


def _mid_body(p_ref, c_ref, x_ref, w1r_ref, b1l_ref, w2l_ref,
              h1_ref, z2_ref):
    cnt = jnp.maximum(c_ref[0, :, :1] + c_ref[1, :, :1], 1.0)
    agg = (p_ref[0] + p_ref[1]) / cnt
    h1 = jnp.maximum(
        agg + b1l_ref[...]
        + jnp.dot(x_ref[...], w1r_ref[...], preferred_element_type=jnp.float32),
        0.0)
    h1_ref[...] = h1
    z2_ref[...] = jnp.dot(h1, w2l_ref[...], preferred_element_type=jnp.float32)


_mid = pl.pallas_call(
    _mid_body,
    grid=(N // BLK,),
    in_specs=[
        pl.BlockSpec((2, BLK, D_H), lambda i: (0, i, 0)),
        pl.BlockSpec((2, BLK, 16), lambda i: (0, i, 0)),
        pl.BlockSpec((BLK, D_IN), lambda i: (i, 0)),
        pl.BlockSpec((D_IN, D_H), lambda i: (0, 0)),
        pl.BlockSpec((1, D_H), lambda i: (0, 0)),
        pl.BlockSpec((D_H, D_H), lambda i: (0, 0)),
    ],
    out_specs=[
        pl.BlockSpec((BLK, D_H), lambda i: (i, 0)),
        pl.BlockSpec((BLK, D_H), lambda i: (i, 0)),
    ],
    out_shape=[
        jax.ShapeDtypeStruct((N, D_H), jnp.float32),
        jax.ShapeDtypeStruct((N, D_H), jnp.float32),
    ],
)


def _fin_body(p_ref, c_ref, h1_ref, w2r_ref, b2l_ref,
              wp1_ref, bp1_ref, wp2_ref, bp2_ref, o_ref):
    cnt = jnp.maximum(c_ref[0, :, :1] + c_ref[1, :, :1], 1.0)
    agg = (p_ref[0] + p_ref[1]) / cnt
    h2 = jnp.maximum(
        agg + b2l_ref[...]
        + jnp.dot(h1_ref[...], w2r_ref[...],
                  preferred_element_type=jnp.float32),
        0.0)
    p = jnp.maximum(
        jnp.dot(h2, wp1_ref[...], preferred_element_type=jnp.float32)
        + bp1_ref[...],
        0.0)
    o_ref[...] = (jnp.dot(p, wp2_ref[...], preferred_element_type=jnp.float32)
                  + bp2_ref[...])


_fin = pl.pallas_call(
    _fin_body,
    grid=(N // BLK,),
    in_specs=[
        pl.BlockSpec((2, BLK, D_H), lambda i: (0, i, 0)),
        pl.BlockSpec((2, BLK, 16), lambda i: (0, i, 0)),
        pl.BlockSpec((BLK, D_H), lambda i: (i, 0)),
        pl.BlockSpec((D_H, D_H), lambda i: (0, 0)),
        pl.BlockSpec((1, D_H), lambda i: (0, 0)),
        pl.BlockSpec((D_H, D_H // 2), lambda i: (0, 0)),
        pl.BlockSpec((1, D_H // 2), lambda i: (0, 0)),
        pl.BlockSpec((D_H // 2, 1), lambda i: (0, 0)),
        pl.BlockSpec((1, 1), lambda i: (0, 0)),
    ],
    out_specs=pl.BlockSpec((BLK, 1), lambda i: (i, 0)),
    out_shape=jax.ShapeDtypeStruct((N, 1), jnp.float32),
)


def kernel(x, edge_index, W1l, b1l, W1r, W2l, b2l, W2r, Wp1, bp1, Wp2, bp2):
    edges = edge_index.reshape(2, NW, NCH, CHUNK)
    zrow = jnp.zeros((RPT, D_H), jnp.float32)
    zcnt = jnp.zeros((RPT, 16), jnp.float32)

    b1l2 = b1l.reshape(1, D_H)
    b2l2 = b2l.reshape(1, D_H)
    bp12 = bp1.reshape(1, D_H // 2)
    bp22 = bp2.reshape(1, 1)

    z1 = _mm(x, W1l)
    p1, cnt = _agg_with_counts(z1, edges, zrow, zcnt)
    h1, z2 = _mid(p1, cnt, x, W1r, b1l2, W2l)
    (p2,) = _agg_no_counts(z2, edges, zrow)
    return _fin(p2, cnt, h1, W2r, b2l2, Wp1, bp12, Wp2, bp22)


# speedup vs baseline: 2.4689x; 1.1755x over previous
import functools

import jax
import jax.numpy as jnp
from jax import lax
from jax.experimental import pallas as pl
from jax.experimental.pallas import tpu as pltpu
from jax.experimental.pallas import tpu_sc as plsc

N = 10000
E = 320000
D_IN = 128
D_H = 64

NC = 2
NS = 16
NW = NC * NS

EPW = E // NW
CHUNK = 80
NCH = 125
EPW_PAD = NCH * CHUNK
RPT = 632
N_PAD = NS * RPT

BLK = 2000


def _agg_body(with_counts, *refs):
    if with_counts:
        (z_hbm, edge_hbm, zrow_hbm, zcnt_hbm,
         out_hbm, cnt_hbm,
         src_v, dst_v, rows0, rows1, rows2, ones_v,
         acc_sh, cnt_sh, sem0, sem1, sem2,
         semc0, semc1, semc2) = refs
    else:
        (z_hbm, edge_hbm, zrow_hbm,
         out_hbm,
         src_v, dst_v, rows0, rows1, rows2,
         acc_sh, sem0, sem1, sem2) = refs
        semc0 = semc1 = semc2 = None

    c = lax.axis_index("c")
    s = lax.axis_index("s")
    wid = c * NS + s

    pltpu.sync_copy(edge_hbm.at[0, wid], src_v)
    pltpu.sync_copy(edge_hbm.at[1, wid], dst_v)

    pltpu.sync_copy(zrow_hbm, acc_sh.at[pl.ds(s * RPT, RPT)])
    if with_counts:
        pltpu.sync_copy(zcnt_hbm, cnt_sh.at[pl.ds(s * RPT, RPT)])

        def _fill_ones(i, carry):
            ones_v[i, :] = jnp.ones((16,), jnp.float32)
            return carry
        lax.fori_loop(0, CHUNK, _fill_ones, 0)

    plsc.subcore_barrier()

    bufs = (rows0, rows1, rows2)
    sems = (sem0, sem1, sem2)
    semcs = (semc0, semc1, semc2)
    for b in range(3):
        pltpu.async_copy(z_hbm.at[src_v.at[b]], bufs[b], sems[b])

    def _step(cidx, b, last=False):
        rows, sem, semc = bufs[b], sems[b], semcs[b]
        pltpu.make_async_copy(z_hbm.at[src_v.at[0]], rows, sem).wait()
        pltpu.sync_copy(rows, acc_sh.at[dst_v.at[cidx]], add=True)
        if with_counts:
            @pl.when(cidx >= 3)
            def _():
                pltpu.make_async_copy(ones_v, cnt_sh.at[dst_v.at[0]],
                                      semc).wait()
            pltpu.async_copy(ones_v, cnt_sh.at[dst_v.at[cidx]], semc,
                             add=True)
        if last:
            return

        @pl.when(cidx + 3 < NCH)
        def _():
            pltpu.async_copy(z_hbm.at[src_v.at[cidx + 3]], rows, sem)

    def _loop(g, carry):
        _step(3 * g, 0)
        _step(3 * g + 1, 1)
        _step(3 * g + 2, 2)
        return carry
    lax.fori_loop(0, NCH // 3, _loop, 0)
    _step(NCH - 2, 0, last=True)
    _step(NCH - 1, 1, last=True)
    if with_counts:
        for b in range(3):
            pltpu.make_async_copy(ones_v, cnt_sh.at[dst_v.at[0]],
                                  semcs[b]).wait()

    plsc.subcore_barrier()

    pltpu.sync_copy(acc_sh.at[pl.ds(s * RPT, RPT)],
                    out_hbm.at[c, pl.ds(s * RPT, RPT)])
    if with_counts:
        pltpu.sync_copy(cnt_sh.at[pl.ds(s * RPT, RPT)],
                        cnt_hbm.at[c, pl.ds(s * RPT, RPT)])


def _make_agg(with_counts):
    mesh = plsc.VectorSubcoreMesh(core_axis_name="c", subcore_axis_name="s")
    out_type = [jax.ShapeDtypeStruct((NC, N_PAD, D_H), jnp.float32)]
    scratch = [
        pltpu.VMEM((NCH, CHUNK), jnp.int32),
        pltpu.VMEM((NCH, CHUNK), jnp.int32),
        pltpu.VMEM((CHUNK, D_H), jnp.float32),
        pltpu.VMEM((CHUNK, D_H), jnp.float32),
        pltpu.VMEM((CHUNK, D_H), jnp.float32),
    ]
    if with_counts:
        out_type.append(jax.ShapeDtypeStruct((NC, N_PAD, 16), jnp.float32))
        scratch.append(pltpu.VMEM((CHUNK, 16), jnp.float32))
    scratch.append(pltpu.VMEM_SHARED((N_PAD, D_H), jnp.float32))
    if with_counts:
        scratch.append(pltpu.VMEM_SHARED((N_PAD, 16), jnp.float32))
    scratch += [pltpu.SemaphoreType.DMA] * 3
    if with_counts:
        scratch += [pltpu.SemaphoreType.DMA] * 3
    return pl.kernel(
        functools.partial(_agg_body, with_counts),
        out_type=out_type,
        mesh=mesh,
        scratch_types=scratch,
        compiler_params=pltpu.CompilerParams(use_tc_tiling_on_sc=False),
    )


_agg_with_counts = _make_agg(True)
_agg_no_counts = _make_agg(False)


def _mm_body(x_ref, w_ref, o_ref):
    o_ref[...] = jnp.dot(x_ref[...], w_ref[...],
                         preferred_element_type=jnp.float32)


_mm = pl.pallas_call(
    _mm_body,
    grid=(N // BLK,),
    in_specs=[
        pl.BlockSpec((BLK, D_IN), lambda i: (i, 0)),
        pl.BlockSpec((D_IN, D_H), lambda i: (0, 0)),
    ],
    out_specs=pl.BlockSpec((BLK, D_H), lambda i: (i, 0)),
    out_shape=jax.ShapeDtypeStruct((N, D_H), jnp.float32),
)


def _mid_body(p_ref, c_ref, x_ref, w1r_ref, b1l_ref, w2l_ref,
              h1_ref, z2_ref):
    cnt = jnp.maximum(c_ref[0, :, :1] + c_ref[1, :, :1], 1.0)
    agg = (p_ref[0] + p_ref[1]) / cnt
    h1 = jnp.maximum(
        agg + b1l_ref[...]
        + jnp.dot(x_ref[...], w1r_ref[...], preferred_element_type=jnp.float32),
        0.0)
    h1_ref[...] = h1
    z2_ref[...] = jnp.dot(h1, w2l_ref[...], preferred_element_type=jnp.float32)


_mid = pl.pallas_call(
    _mid_body,
    grid=(N // BLK,),
    in_specs=[
        pl.BlockSpec((2, BLK, D_H), lambda i: (0, i, 0)),
        pl.BlockSpec((2, BLK, 16), lambda i: (0, i, 0)),
        pl.BlockSpec((BLK, D_IN), lambda i: (i, 0)),
        pl.BlockSpec((D_IN, D_H), lambda i: (0, 0)),
        pl.BlockSpec((1, D_H), lambda i: (0, 0)),
        pl.BlockSpec((D_H, D_H), lambda i: (0, 0)),
    ],
    out_specs=[
        pl.BlockSpec((BLK, D_H), lambda i: (i, 0)),
        pl.BlockSpec((BLK, D_H), lambda i: (i, 0)),
    ],
    out_shape=[
        jax.ShapeDtypeStruct((N, D_H), jnp.float32),
        jax.ShapeDtypeStruct((N, D_H), jnp.float32),
    ],
)


def _fin_body(p_ref, c_ref, h1_ref, w2r_ref, b2l_ref,
              wp1_ref, bp1_ref, wp2_ref, bp2_ref, o_ref):
    cnt = jnp.maximum(c_ref[0, :, :1] + c_ref[1, :, :1], 1.0)
    agg = (p_ref[0] + p_ref[1]) / cnt
    h2 = jnp.maximum(
        agg + b2l_ref[...]
        + jnp.dot(h1_ref[...], w2r_ref[...],
                  preferred_element_type=jnp.float32),
        0.0)
    p = jnp.maximum(
        jnp.dot(h2, wp1_ref[...], preferred_element_type=jnp.float32)
        + bp1_ref[...],
        0.0)
    o_ref[...] = (jnp.dot(p, wp2_ref[...], preferred_element_type=jnp.float32)
                  + bp2_ref[...])


_fin = pl.pallas_call(
    _fin_body,
    grid=(N // BLK,),
    in_specs=[
        pl.BlockSpec((2, BLK, D_H), lambda i: (0, i, 0)),
        pl.BlockSpec((2, BLK, 16), lambda i: (0, i, 0)),
        pl.BlockSpec((BLK, D_H), lambda i: (i, 0)),
        pl.BlockSpec((D_H, D_H), lambda i: (0, 0)),
        pl.BlockSpec((1, D_H), lambda i: (0, 0)),
        pl.BlockSpec((D_H, D_H // 2), lambda i: (0, 0)),
        pl.BlockSpec((1, D_H // 2), lambda i: (0, 0)),
        pl.BlockSpec((D_H // 2, 1), lambda i: (0, 0)),
        pl.BlockSpec((1, 1), lambda i: (0, 0)),
    ],
    out_specs=pl.BlockSpec((BLK, 1), lambda i: (i, 0)),
    out_shape=jax.ShapeDtypeStruct((N, 1), jnp.float32),
)


def kernel(x, edge_index, W1l, b1l, W1r, W2l, b2l, W2r, Wp1, bp1, Wp2, bp2):
    edges = edge_index.reshape(2, NW, NCH, CHUNK)
    zrow = jnp.zeros((RPT, D_H), jnp.float32)
    zcnt = jnp.zeros((RPT, 16), jnp.float32)

    b1l2 = b1l.reshape(1, D_H)
    b2l2 = b2l.reshape(1, D_H)
    bp12 = bp1.reshape(1, D_H // 2)
    bp22 = bp2.reshape(1, 1)

    z1 = _mm(x, W1l)
    p1, cnt = _agg_with_counts(z1, edges, zrow, zcnt)
    h1, z2 = _mid(p1, cnt, x, W1r, b1l2, W2l)
    (p2,) = _agg_no_counts(z2, edges, zrow)
    return _fin(p2, cnt, h1, W2r, b2l2, Wp1, bp12, Wp2, bp22)


# speedup vs baseline: 2.6347x; 1.0672x over previous
import functools

import jax
import jax.numpy as jnp
from jax import lax
from jax.experimental import pallas as pl
from jax.experimental.pallas import tpu as pltpu
from jax.experimental.pallas import tpu_sc as plsc

N = 10000
E = 320000
D_IN = 128
D_H = 64

NC = 2
NS = 16
NW = NC * NS

EPW = E // NW
CHUNK = 80
NCH = 125
NBUF = 4
RPT = 632
N_PAD = NS * RPT

BLK = 2000


def _agg_body(with_counts, *refs):
    if with_counts:
        (z_hbm, edge_hbm, zrow_hbm, zcnt_hbm,
         out_hbm, cnt_hbm,
         src_v, dst_v, *rest) = refs
        (*bufs, ones_v, acc_sh, cnt_sh) = rest[:NBUF + 3]
        sems = rest[NBUF + 3:2 * NBUF + 3]
        semcs = rest[2 * NBUF + 3:]
    else:
        (z_hbm, edge_hbm, zrow_hbm,
         out_hbm,
         src_v, dst_v, *rest) = refs
        bufs = rest[:NBUF]
        acc_sh = rest[NBUF]
        sems = rest[NBUF + 1:]
        semcs = None

    c = lax.axis_index("c")
    s = lax.axis_index("s")
    wid = c * NS + s

    pltpu.sync_copy(edge_hbm.at[0, wid], src_v)
    pltpu.sync_copy(edge_hbm.at[1, wid], dst_v)

    pltpu.sync_copy(zrow_hbm, acc_sh.at[pl.ds(s * RPT, RPT)])
    if with_counts:
        pltpu.sync_copy(zcnt_hbm, cnt_sh.at[pl.ds(s * RPT, RPT)])

        def _fill_ones(i, carry):
            ones_v[i, :] = jnp.ones((16,), jnp.float32)
            return carry
        lax.fori_loop(0, CHUNK, _fill_ones, 0)

    plsc.subcore_barrier()

    for b in range(NBUF):
        pltpu.async_copy(z_hbm.at[src_v.at[b]], bufs[b], sems[b])

    def _step(cidx, b, last=False):
        rows, sem = bufs[b], sems[b]
        pltpu.make_async_copy(z_hbm.at[src_v.at[0]], rows, sem).wait()
        pltpu.sync_copy(rows, acc_sh.at[dst_v.at[cidx]], add=True)
        if with_counts:
            semc = semcs[b]

            @pl.when(cidx >= NBUF)
            def _():
                pltpu.make_async_copy(ones_v, cnt_sh.at[dst_v.at[0]],
                                      semc).wait()
            pltpu.async_copy(ones_v, cnt_sh.at[dst_v.at[cidx]], semc,
                             add=True)
        if last:
            return

        @pl.when(cidx + NBUF < NCH)
        def _():
            pltpu.async_copy(z_hbm.at[src_v.at[cidx + NBUF]], rows, sem)

    def _loop(g, carry):
        for b in range(NBUF):
            _step(NBUF * g + b, b)
        return carry
    lax.fori_loop(0, NCH // NBUF, _loop, 0)
    for r in range(NCH - (NCH // NBUF) * NBUF):
        _step((NCH // NBUF) * NBUF + r, r, last=True)
    if with_counts:
        for b in range(NBUF):
            pltpu.make_async_copy(ones_v, cnt_sh.at[dst_v.at[0]],
                                  semcs[b]).wait()

    plsc.subcore_barrier()

    pltpu.sync_copy(acc_sh.at[pl.ds(s * RPT, RPT)],
                    out_hbm.at[c, pl.ds(s * RPT, RPT)])
    if with_counts:
        pltpu.sync_copy(cnt_sh.at[pl.ds(s * RPT, RPT)],
                        cnt_hbm.at[c, pl.ds(s * RPT, RPT)])


def _make_agg(with_counts):
    mesh = plsc.VectorSubcoreMesh(core_axis_name="c", subcore_axis_name="s")
    out_type = [jax.ShapeDtypeStruct((NC, N_PAD, D_H), jnp.float32)]
    scratch = [
        pltpu.VMEM((NCH, CHUNK), jnp.int32),
        pltpu.VMEM((NCH, CHUNK), jnp.int32),
    ]
    scratch += [pltpu.VMEM((CHUNK, D_H), jnp.float32)] * NBUF
    if with_counts:
        out_type.append(jax.ShapeDtypeStruct((NC, N_PAD, 16), jnp.float32))
        scratch.append(pltpu.VMEM((CHUNK, 16), jnp.float32))
    scratch.append(pltpu.VMEM_SHARED((N_PAD, D_H), jnp.float32))
    if with_counts:
        scratch.append(pltpu.VMEM_SHARED((N_PAD, 16), jnp.float32))
    scratch += [pltpu.SemaphoreType.DMA] * NBUF
    if with_counts:
        scratch += [pltpu.SemaphoreType.DMA] * NBUF
    return pl.kernel(
        functools.partial(_agg_body, with_counts),
        out_type=out_type,
        mesh=mesh,
        scratch_types=scratch,
        compiler_params=pltpu.CompilerParams(use_tc_tiling_on_sc=False),
    )


_agg_with_counts = _make_agg(True)
_agg_no_counts = _make_agg(False)


def _mm_body(x_ref, w_ref, o_ref):
    o_ref[...] = jnp.dot(x_ref[...], w_ref[...],
                         preferred_element_type=jnp.float32)


_mm = pl.pallas_call(
    _mm_body,
    grid=(N // BLK,),
    in_specs=[
        pl.BlockSpec((BLK, D_IN), lambda i: (i, 0)),
        pl.BlockSpec((D_IN, D_H), lambda i: (0, 0)),
    ],
    out_specs=pl.BlockSpec((BLK, D_H), lambda i: (i, 0)),
    out_shape=jax.ShapeDtypeStruct((N, D_H), jnp.float32),
)


def _mid_body(p_ref, c_ref, x_ref, w1r_ref, b1l_ref, w2l_ref,
              h1_ref, z2_ref):
    cnt = jnp.maximum(c_ref[0, :, :1] + c_ref[1, :, :1], 1.0)
    agg = (p_ref[0] + p_ref[1]) / cnt
    h1 = jnp.maximum(
        agg + b1l_ref[...]
        + jnp.dot(x_ref[...], w1r_ref[...], preferred_element_type=jnp.float32),
        0.0)
    h1_ref[...] = h1
    z2_ref[...] = jnp.dot(h1, w2l_ref[...], preferred_element_type=jnp.float32)


_mid = pl.pallas_call(
    _mid_body,
    grid=(N // BLK,),
    in_specs=[
        pl.BlockSpec((2, BLK, D_H), lambda i: (0, i, 0)),
        pl.BlockSpec((2, BLK, 16), lambda i: (0, i, 0)),
        pl.BlockSpec((BLK, D_IN), lambda i: (i, 0)),
        pl.BlockSpec((D_IN, D_H), lambda i: (0, 0)),
        pl.BlockSpec((1, D_H), lambda i: (0, 0)),
        pl.BlockSpec((D_H, D_H), lambda i: (0, 0)),
    ],
    out_specs=[
        pl.BlockSpec((BLK, D_H), lambda i: (i, 0)),
        pl.BlockSpec((BLK, D_H), lambda i: (i, 0)),
    ],
    out_shape=[
        jax.ShapeDtypeStruct((N, D_H), jnp.float32),
        jax.ShapeDtypeStruct((N, D_H), jnp.float32),
    ],
)


def _fin_body(p_ref, c_ref, h1_ref, w2r_ref, b2l_ref,
              wp1_ref, bp1_ref, wp2_ref, bp2_ref, o_ref):
    cnt = jnp.maximum(c_ref[0, :, :1] + c_ref[1, :, :1], 1.0)
    agg = (p_ref[0] + p_ref[1]) / cnt
    h2 = jnp.maximum(
        agg + b2l_ref[...]
        + jnp.dot(h1_ref[...], w2r_ref[...],
                  preferred_element_type=jnp.float32),
        0.0)
    p = jnp.maximum(
        jnp.dot(h2, wp1_ref[...], preferred_element_type=jnp.float32)
        + bp1_ref[...],
        0.0)
    o_ref[...] = (jnp.dot(p, wp2_ref[...], preferred_element_type=jnp.float32)
                  + bp2_ref[...])


_fin = pl.pallas_call(
    _fin_body,
    grid=(N // BLK,),
    in_specs=[
        pl.BlockSpec((2, BLK, D_H), lambda i: (0, i, 0)),
        pl.BlockSpec((2, BLK, 16), lambda i: (0, i, 0)),
        pl.BlockSpec((BLK, D_H), lambda i: (i, 0)),
        pl.BlockSpec((D_H, D_H), lambda i: (0, 0)),
        pl.BlockSpec((1, D_H), lambda i: (0, 0)),
        pl.BlockSpec((D_H, D_H // 2), lambda i: (0, 0)),
        pl.BlockSpec((1, D_H // 2), lambda i: (0, 0)),
        pl.BlockSpec((D_H // 2, 1), lambda i: (0, 0)),
        pl.BlockSpec((1, 1), lambda i: (0, 0)),
    ],
    out_specs=pl.BlockSpec((BLK, 1), lambda i: (i, 0)),
    out_shape=jax.ShapeDtypeStruct((N, 1), jnp.float32),
)


def kernel(x, edge_index, W1l, b1l, W1r, W2l, b2l, W2r, Wp1, bp1, Wp2, bp2):
    edges = edge_index.reshape(2, NW, NCH, CHUNK)
    zrow = jnp.zeros((RPT, D_H), jnp.float32)
    zcnt = jnp.zeros((RPT, 16), jnp.float32)

    b1l2 = b1l.reshape(1, D_H)
    b2l2 = b2l.reshape(1, D_H)
    bp12 = bp1.reshape(1, D_H // 2)
    bp22 = bp2.reshape(1, 1)

    z1 = _mm(x, W1l)
    p1, cnt = _agg_with_counts(z1, edges, zrow, zcnt)
    h1, z2 = _mid(p1, cnt, x, W1r, b1l2, W2l)
    (p2,) = _agg_no_counts(z2, edges, zrow)
    return _fin(p2, cnt, h1, W2r, b2l2, Wp1, bp12, Wp2, bp22)


# speedup vs baseline: 2.6529x; 1.0069x over previous
import functools

import jax
import jax.numpy as jnp
from jax import lax
from jax.experimental import pallas as pl
from jax.experimental.pallas import tpu as pltpu
from jax.experimental.pallas import tpu_sc as plsc

N = 10000
E = 320000
D_IN = 128
D_H = 64

NC = 2
NS = 16
NW = NC * NS

EPW = E // NW
CHUNK = 80
NCH = 125
NBUF = 6
RPT = 632
N_PAD = NS * RPT

BLK = 2000


def _agg_body(with_counts, *refs):
    if with_counts:
        (z_hbm, edge_hbm, zrow_hbm, zcnt_hbm,
         out_hbm, cnt_hbm,
         src_v, dst_v, *rest) = refs
        (*bufs, ones_v, acc_sh, cnt_sh) = rest[:NBUF + 3]
        sems = rest[NBUF + 3:2 * NBUF + 3]
        semcs = rest[2 * NBUF + 3:]
    else:
        (z_hbm, edge_hbm, zrow_hbm,
         out_hbm,
         src_v, dst_v, *rest) = refs
        bufs = rest[:NBUF]
        acc_sh = rest[NBUF]
        sems = rest[NBUF + 1:]
        semcs = None

    c = lax.axis_index("c")
    s = lax.axis_index("s")
    wid = c * NS + s

    pltpu.sync_copy(edge_hbm.at[0, wid], src_v)
    pltpu.sync_copy(edge_hbm.at[1, wid], dst_v)

    pltpu.sync_copy(zrow_hbm, acc_sh.at[pl.ds(s * RPT, RPT)])
    if with_counts:
        pltpu.sync_copy(zcnt_hbm, cnt_sh.at[pl.ds(s * RPT, RPT)])

        def _fill_ones(i, carry):
            ones_v[i, :] = jnp.ones((16,), jnp.float32)
            return carry
        lax.fori_loop(0, CHUNK, _fill_ones, 0)

    plsc.subcore_barrier()

    for b in range(NBUF):
        pltpu.async_copy(z_hbm.at[src_v.at[b]], bufs[b], sems[b])

    def _step(cidx, b, last=False):
        rows, sem = bufs[b], sems[b]
        pltpu.make_async_copy(z_hbm.at[src_v.at[0]], rows, sem).wait()
        pltpu.sync_copy(rows, acc_sh.at[dst_v.at[cidx]], add=True)
        if with_counts:
            semc = semcs[b]

            @pl.when(cidx >= NBUF)
            def _():
                pltpu.make_async_copy(ones_v, cnt_sh.at[dst_v.at[0]],
                                      semc).wait()
            pltpu.async_copy(ones_v, cnt_sh.at[dst_v.at[cidx]], semc,
                             add=True)
        if last:
            return

        @pl.when(cidx + NBUF < NCH)
        def _():
            pltpu.async_copy(z_hbm.at[src_v.at[cidx + NBUF]], rows, sem)

    def _loop(g, carry):
        for b in range(NBUF):
            _step(NBUF * g + b, b)
        return carry
    lax.fori_loop(0, NCH // NBUF, _loop, 0)
    for r in range(NCH - (NCH // NBUF) * NBUF):
        _step((NCH // NBUF) * NBUF + r, r, last=True)
    if with_counts:
        for b in range(NBUF):
            pltpu.make_async_copy(ones_v, cnt_sh.at[dst_v.at[0]],
                                  semcs[b]).wait()

    plsc.subcore_barrier()

    pltpu.sync_copy(acc_sh.at[pl.ds(s * RPT, RPT)],
                    out_hbm.at[c, pl.ds(s * RPT, RPT)])
    if with_counts:
        pltpu.sync_copy(cnt_sh.at[pl.ds(s * RPT, RPT)],
                        cnt_hbm.at[c, pl.ds(s * RPT, RPT)])


def _make_agg(with_counts):
    mesh = plsc.VectorSubcoreMesh(core_axis_name="c", subcore_axis_name="s")
    out_type = [jax.ShapeDtypeStruct((NC, N_PAD, D_H), jnp.float32)]
    scratch = [
        pltpu.VMEM((NCH, CHUNK), jnp.int32),
        pltpu.VMEM((NCH, CHUNK), jnp.int32),
    ]
    scratch += [pltpu.VMEM((CHUNK, D_H), jnp.float32)] * NBUF
    if with_counts:
        out_type.append(jax.ShapeDtypeStruct((NC, N_PAD, 16), jnp.float32))
        scratch.append(pltpu.VMEM((CHUNK, 16), jnp.float32))
    scratch.append(pltpu.VMEM_SHARED((N_PAD, D_H), jnp.float32))
    if with_counts:
        scratch.append(pltpu.VMEM_SHARED((N_PAD, 16), jnp.float32))
    scratch += [pltpu.SemaphoreType.DMA] * NBUF
    if with_counts:
        scratch += [pltpu.SemaphoreType.DMA] * NBUF
    return pl.kernel(
        functools.partial(_agg_body, with_counts),
        out_type=out_type,
        mesh=mesh,
        scratch_types=scratch,
        compiler_params=pltpu.CompilerParams(use_tc_tiling_on_sc=False),
    )


_agg_with_counts = _make_agg(True)
_agg_no_counts = _make_agg(False)


def _mm_body(x_ref, w_ref, o_ref):
    o_ref[...] = jnp.dot(x_ref[...], w_ref[...],
                         preferred_element_type=jnp.float32)


_mm = pl.pallas_call(
    _mm_body,
    grid=(N // BLK,),
    in_specs=[
        pl.BlockSpec((BLK, D_IN), lambda i: (i, 0)),
        pl.BlockSpec((D_IN, D_H), lambda i: (0, 0)),
    ],
    out_specs=pl.BlockSpec((BLK, D_H), lambda i: (i, 0)),
    out_shape=jax.ShapeDtypeStruct((N, D_H), jnp.float32),
)


def _mid_body(p_ref, c_ref, x_ref, w1r_ref, b1l_ref, w2l_ref,
              h1_ref, z2_ref):
    cnt = jnp.maximum(c_ref[0, :, :1] + c_ref[1, :, :1], 1.0)
    agg = (p_ref[0] + p_ref[1]) / cnt
    h1 = jnp.maximum(
        agg + b1l_ref[...]
        + jnp.dot(x_ref[...], w1r_ref[...], preferred_element_type=jnp.float32),
        0.0)
    h1_ref[...] = h1
    z2_ref[...] = jnp.dot(h1, w2l_ref[...], preferred_element_type=jnp.float32)


_mid = pl.pallas_call(
    _mid_body,
    grid=(N // BLK,),
    in_specs=[
        pl.BlockSpec((2, BLK, D_H), lambda i: (0, i, 0)),
        pl.BlockSpec((2, BLK, 16), lambda i: (0, i, 0)),
        pl.BlockSpec((BLK, D_IN), lambda i: (i, 0)),
        pl.BlockSpec((D_IN, D_H), lambda i: (0, 0)),
        pl.BlockSpec((1, D_H), lambda i: (0, 0)),
        pl.BlockSpec((D_H, D_H), lambda i: (0, 0)),
    ],
    out_specs=[
        pl.BlockSpec((BLK, D_H), lambda i: (i, 0)),
        pl.BlockSpec((BLK, D_H), lambda i: (i, 0)),
    ],
    out_shape=[
        jax.ShapeDtypeStruct((N, D_H), jnp.float32),
        jax.ShapeDtypeStruct((N, D_H), jnp.float32),
    ],
)


def _fin_body(p_ref, c_ref, h1_ref, w2r_ref, b2l_ref,
              wp1_ref, bp1_ref, wp2_ref, bp2_ref, o_ref):
    cnt = jnp.maximum(c_ref[0, :, :1] + c_ref[1, :, :1], 1.0)
    agg = (p_ref[0] + p_ref[1]) / cnt
    h2 = jnp.maximum(
        agg + b2l_ref[...]
        + jnp.dot(h1_ref[...], w2r_ref[...],
                  preferred_element_type=jnp.float32),
        0.0)
    p = jnp.maximum(
        jnp.dot(h2, wp1_ref[...], preferred_element_type=jnp.float32)
        + bp1_ref[...],
        0.0)
    o_ref[...] = (jnp.dot(p, wp2_ref[...], preferred_element_type=jnp.float32)
                  + bp2_ref[...])


_fin = pl.pallas_call(
    _fin_body,
    grid=(N // BLK,),
    in_specs=[
        pl.BlockSpec((2, BLK, D_H), lambda i: (0, i, 0)),
        pl.BlockSpec((2, BLK, 16), lambda i: (0, i, 0)),
        pl.BlockSpec((BLK, D_H), lambda i: (i, 0)),
        pl.BlockSpec((D_H, D_H), lambda i: (0, 0)),
        pl.BlockSpec((1, D_H), lambda i: (0, 0)),
        pl.BlockSpec((D_H, D_H // 2), lambda i: (0, 0)),
        pl.BlockSpec((1, D_H // 2), lambda i: (0, 0)),
        pl.BlockSpec((D_H // 2, 1), lambda i: (0, 0)),
        pl.BlockSpec((1, 1), lambda i: (0, 0)),
    ],
    out_specs=pl.BlockSpec((BLK, 1), lambda i: (i, 0)),
    out_shape=jax.ShapeDtypeStruct((N, 1), jnp.float32),
)


def kernel(x, edge_index, W1l, b1l, W1r, W2l, b2l, W2r, Wp1, bp1, Wp2, bp2):
    edges = edge_index.reshape(2, NW, NCH, CHUNK)
    zrow = jnp.zeros((RPT, D_H), jnp.float32)
    zcnt = jnp.zeros((RPT, 16), jnp.float32)

    b1l2 = b1l.reshape(1, D_H)
    b2l2 = b2l.reshape(1, D_H)
    bp12 = bp1.reshape(1, D_H // 2)
    bp22 = bp2.reshape(1, 1)

    z1 = _mm(x, W1l)
    p1, cnt = _agg_with_counts(z1, edges, zrow, zcnt)
    h1, z2 = _mid(p1, cnt, x, W1r, b1l2, W2l)
    (p2,) = _agg_no_counts(z2, edges, zrow)
    return _fin(p2, cnt, h1, W2r, b2l2, Wp1, bp12, Wp2, bp22)
